# baseline (speedup 1.0000x reference)
import functools

import jax
import jax.numpy as jnp
from jax import lax
from jax.experimental import pallas as pl
from jax.experimental.pallas import tpu as pltpu
from jax.experimental.pallas import tpu_sc as plsc

_B = 16384
_D = 64
_NC = 2
_NS = 16
_NW = _NC * _NS
_BW = _B // _NW
_CH = 128
_NCH = _BW // _CH
_L = 16


def _mf_body(uid, iid, uemb, iemb, ubias, ibias, gbias, out,
             uidx, iidx, urows, irows, ub, ib, gb, outv, pacc, sem):
    c = lax.axis_index("c")
    s = lax.axis_index("s")
    base = (s * _NC + c) * _BW

    pltpu.sync_copy(uid.at[pl.ds(base, _BW)], uidx)
    pltpu.sync_copy(iid.at[pl.ds(base, _BW)], iidx)
    pltpu.sync_copy(gbias, gb)

    cps = []
    for k in range(_NCH):
        sl = pl.ds(k * _CH, _CH)
        cps.append(pltpu.async_copy(uemb.at[uidx.at[sl]], urows.at[sl], sem))
        cps.append(pltpu.async_copy(iemb.at[iidx.at[sl]], irows.at[sl], sem))
        cps.append(pltpu.async_copy(ubias.at[uidx.at[sl]], ub.at[sl], sem))
        cps.append(pltpu.async_copy(ibias.at[iidx.at[sl]], ib.at[sl], sem))
    for cp in cps:
        cp.wait()

    gvec = gb[...]
    rowsel = lax.iota(jnp.int32, _L) * _L

    def group(g, carry):
        osl = pl.ds(g * _L, _L)
        for r in range(_L):
            row = g * _L + r
            acc = None
            for cc in range(_D // _L):
                csl = pl.ds(cc * _L, _L)
                p = urows[row, csl] * irows[row, csl]
                acc = p if acc is None else acc + p
            pacc[pl.ds(r * _L, _L)] = acc
        tot = (ub[osl] + ib[osl]) + gvec
        for cc in range(_L):
            tot = tot + plsc.load_gather(pacc, [rowsel + cc])
        outv[osl] = tot
        return carry

    lax.fori_loop(0, _BW // _L, group, 0)
    pltpu.sync_copy(outv, out.at[pl.ds(base, _BW)])


def kernel(user_ids, item_ids, user_emb, item_emb, user_bias, item_bias, global_bias):
    uid = user_ids.astype(jnp.int32)
    iid = item_ids.astype(jnp.int32)
    ubias = user_bias.reshape(-1)
    ibias = item_bias.reshape(-1)
    gb16 = jnp.broadcast_to(global_bias.astype(jnp.float32), (_L,))
    mesh = plsc.VectorSubcoreMesh(core_axis_name="c", subcore_axis_name="s")
    f = pl.kernel(
        _mf_body,
        mesh=mesh,
        compiler_params=pltpu.CompilerParams(
            needs_layout_passes=False, use_tc_tiling_on_sc=False),
        out_type=jax.ShapeDtypeStruct((_B,), jnp.float32),
        scratch_types=[
            pltpu.VMEM((_BW,), jnp.int32),
            pltpu.VMEM((_BW,), jnp.int32),
            pltpu.VMEM((_BW, _D), jnp.float32),
            pltpu.VMEM((_BW, _D), jnp.float32),
            pltpu.VMEM((_BW,), jnp.float32),
            pltpu.VMEM((_BW,), jnp.float32),
            pltpu.VMEM((_L,), jnp.float32),
            pltpu.VMEM((_BW,), jnp.float32),
            pltpu.VMEM((_L * _L,), jnp.float32),
            pltpu.SemaphoreType.DMA,
        ],
    )
    return f(uid, iid, user_emb, item_emb, ubias, ibias, gb16)
